# baseline (speedup 1.0000x reference)
import functools

import jax
import jax.numpy as jnp
from jax import lax
from jax.experimental import pallas as pl
from jax.experimental.pallas import tpu as pltpu
from jax.experimental.pallas import tpu_sc as plsc

EMBED = 64
LANES = 16
NUM_WORKERS = 32
HIST = 200
PAD = 208
NB = 2
SCALE = 8.0
NG = PAD // LANES


def _body(x_hbm, tab_hbm, out_hbm, *scratch):
    idx = scratch[0:NB]
    idx2 = scratch[NB:2 * NB]
    rows = scratch[2 * NB:3 * NB]
    cmp = scratch[3 * NB:4 * NB]
    gsem = scratch[4 * NB:5 * NB]
    ssem = scratch[5 * NB:6 * NB]

    wid = lax.axis_index("s") * 2 + lax.axis_index("c")
    n_rows = x_hbm.shape[0] // HIST
    per_w = n_rows // NUM_WORKERS
    base = wid * per_w
    lane = lax.iota(jnp.int32, LANES)

    def fill(c, b):
        start = (base + c) * HIST
        pltpu.sync_copy(x_hbm.at[pl.ds(start, HIST)], idx[b].at[pl.ds(0, HIST)])
        for g in range(NG):
            sl = pl.ds(g * LANES, LANES)
            v = idx[b][sl] >> 1
            if g == NG - 1:
                v = jnp.where(lane < HIST - (NG - 1) * LANES, v, 0)
            idx2[b][sl] = v
        pltpu.async_copy(tab_hbm.at[idx2[b]], rows[b], gsem[b])

    def wait_gather(b):
        pltpu.make_async_copy(
            tab_hbm.at[pl.ds(0, PAD)], rows[b], gsem[b]
        ).wait()

    def store(c, b):
        pltpu.async_copy(
            cmp[b].at[pl.ds(0, HIST)], out_hbm.at[base + c], ssem[b]
        )

    def wait_store(b):
        pltpu.make_async_copy(
            cmp[b].at[pl.ds(0, HIST)], out_hbm.at[base], ssem[b]
        ).wait()

    def select(b):
        rb, cb, ib = rows[b], cmp[b], idx[b]
        for g in range(NG):
            sl = pl.ds(g * LANES, LANES)
            rid = lane + g * LANES
            off = (ib[sl] & 1) * EMBED

            @plsc.parallel_loop(0, EMBED, step=1, unroll=8)
            def _(col):
                vals = plsc.load_gather(rb, [rid, off + col])
                plsc.store_scatter(cb, [rid, lane * 0 + col], vals * SCALE)

    fill(0, 0)

    def group(g, carry):
        for b in range(NB):
            c = g * NB + b
            br = (b + 1) % NB
            cr = c + 1

            @pl.when(cr < per_w)
            def _():
                @pl.when(c >= 1)
                def _():
                    wait_store(br)

                fill(cr, br)

            wait_gather(b)
            select(b)
            store(c, b)
        return carry

    lax.fori_loop(0, per_w // NB, group, 0)
    for b in range(NB):
        wait_store(b)


def kernel(x, table):
    b, h = x.shape
    n = b * h
    xf = x.reshape(n).astype(jnp.int32)
    tab2 = jnp.reshape(table, (table.shape[0] // 2, 2 * EMBED))

    mesh = plsc.VectorSubcoreMesh(core_axis_name="c", subcore_axis_name="s")
    scratch = (
        [pltpu.VMEM((PAD,), jnp.int32) for _ in range(2 * NB)]
        + [pltpu.VMEM((PAD, 2 * EMBED), jnp.float32) for _ in range(NB)]
        + [pltpu.VMEM((PAD, EMBED), jnp.float32) for _ in range(NB)]
        + [pltpu.SemaphoreType.DMA for _ in range(2 * NB)]
    )
    k = functools.partial(
        pl.kernel,
        out_type=jax.ShapeDtypeStruct((b, h, EMBED), jnp.float32),
        mesh=mesh,
        scratch_types=scratch,
        compiler_params=pltpu.CompilerParams(
            use_tc_tiling_on_sc=True, needs_layout_passes=False
        ),
    )(_body)
    return k(xf, tab2)


# speedup vs baseline: 3.1466x; 3.1466x over previous
import functools

import jax
import jax.numpy as jnp
from jax import lax
from jax.experimental import pallas as pl
from jax.experimental.pallas import tpu as pltpu
from jax.experimental.pallas import tpu_sc as plsc

EMBED = 64
LANES = 16
NUM_WORKERS = 32
HIST = 200
NB = 4
SCALE = 8.0


def _body(x_hbm, tab_hbm, out_hbm, *scratch):
    idx = scratch[0:NB]
    rows = scratch[NB:2 * NB]
    gsem = scratch[2 * NB:3 * NB]
    ssem = scratch[3 * NB:4 * NB]

    wid = lax.axis_index("s") * 2 + lax.axis_index("c")
    n_rows = x_hbm.shape[0] // HIST
    per_w = n_rows // NUM_WORKERS
    base = wid * per_w

    def fill(c, b):
        start = (base + c) * HIST
        pltpu.sync_copy(x_hbm.at[pl.ds(start, HIST)], idx[b])
        pltpu.async_copy(tab_hbm.at[idx[b]], rows[b], gsem[b])

    def wait_gather(b):
        pltpu.make_async_copy(
            tab_hbm.at[pl.ds(0, HIST)], rows[b], gsem[b]
        ).wait()

    def store(c, b):
        pltpu.async_copy(
            rows[b], out_hbm.at[base + c, :, pl.ds(0, EMBED)], ssem[b]
        )

    def wait_store(b):
        pltpu.make_async_copy(
            rows[b], out_hbm.at[base, :, pl.ds(0, EMBED)], ssem[b]
        ).wait()

    def scale(b):
        rb = rows[b]

        @plsc.parallel_loop(0, HIST, step=1, unroll=8)
        def _(r):
            for j in range(EMBED // LANES):
                sl = pl.ds(j * LANES, LANES)
                rb[r, sl] = rb[r, sl] * SCALE

    fill(0, 0)
    fill(1, 1)

    def group(g, carry):
        for b in range(NB):
            c = g * NB + b
            br = (b + 2) % NB
            cr = c + 2

            @pl.when(cr < per_w)
            def _():
                @pl.when(c >= 2)
                def _():
                    wait_store(br)

                fill(cr, br)

            wait_gather(b)
            scale(b)
            store(c, b)
        return carry

    lax.fori_loop(0, per_w // NB, group, 0)
    for b in range(NB):
        wait_store(b)


def kernel(x, table):
    b, h = x.shape
    n = b * h
    xf = x.reshape(n).astype(jnp.int32)

    mesh = plsc.VectorSubcoreMesh(core_axis_name="c", subcore_axis_name="s")
    scratch = (
        [pltpu.VMEM((HIST,), jnp.int32) for _ in range(NB)]
        + [pltpu.VMEM((HIST, EMBED), jnp.float32) for _ in range(NB)]
        + [pltpu.SemaphoreType.DMA for _ in range(2 * NB)]
    )
    k = functools.partial(
        pl.kernel,
        out_type=jax.ShapeDtypeStruct((b, h, 2 * EMBED), jnp.float32),
        mesh=mesh,
        scratch_types=scratch,
        compiler_params=pltpu.CompilerParams(use_tc_tiling_on_sc=False),
    )(_body)
    out = k(xf, table)
    return out[:, :, :EMBED]


# speedup vs baseline: 3.1520x; 1.0017x over previous
import functools

import jax
import jax.numpy as jnp
from jax import lax
from jax.experimental import pallas as pl
from jax.experimental.pallas import tpu as pltpu
from jax.experimental.pallas import tpu_sc as plsc

EMBED = 64
LANES = 16
NUM_WORKERS = 32
HIST = 200
NB = 4
SCALE = 8.0


def _body(x_hbm, tab_hbm, out_hbm, *scratch):
    idx = scratch[0:NB]
    rows = scratch[NB:2 * NB]
    gsem = scratch[2 * NB:3 * NB]
    ssem = scratch[3 * NB:4 * NB]

    wid = lax.axis_index("s") * 2 + lax.axis_index("c")
    n_rows = x_hbm.shape[0] // HIST
    per_w = n_rows // NUM_WORKERS
    base = wid * per_w

    def fill(c, b):
        start = (base + c) * HIST
        pltpu.sync_copy(x_hbm.at[pl.ds(start, HIST)], idx[b])
        pltpu.async_copy(tab_hbm.at[idx[b]], rows[b], gsem[b])

    def wait_gather(b):
        pltpu.make_async_copy(
            tab_hbm.at[pl.ds(0, HIST)], rows[b], gsem[b]
        ).wait()

    def store(c, b):
        pltpu.async_copy(
            rows[b],
            out_hbm.at[pl.ds((base + c) * HIST, HIST), pl.ds(0, EMBED)],
            ssem[b],
        )

    def wait_store(b):
        pltpu.make_async_copy(
            rows[b],
            out_hbm.at[pl.ds(base * HIST, HIST), pl.ds(0, EMBED)],
            ssem[b],
        ).wait()

    def scale(b):
        rb = rows[b]

        @plsc.parallel_loop(0, HIST, step=1, unroll=8)
        def _(r):
            for j in range(EMBED // LANES):
                sl = pl.ds(j * LANES, LANES)
                rb[r, sl] = rb[r, sl] * SCALE

    fill(0, 0)
    fill(1, 1)

    def group(g, carry):
        for b in range(NB):
            c = g * NB + b
            br = (b + 2) % NB
            cr = c + 2

            @pl.when(cr < per_w)
            def _():
                @pl.when(c >= 2)
                def _():
                    wait_store(br)

                fill(cr, br)

            wait_gather(b)
            scale(b)
            store(c, b)
        return carry

    lax.fori_loop(0, per_w // NB, group, 0)
    for b in range(NB):
        wait_store(b)


def kernel(x, table):
    b, h = x.shape
    n = b * h
    xf = x.reshape(n).astype(jnp.int32)

    mesh = plsc.VectorSubcoreMesh(core_axis_name="c", subcore_axis_name="s")
    scratch = (
        [pltpu.VMEM((HIST,), jnp.int32) for _ in range(NB)]
        + [pltpu.VMEM((HIST, EMBED), jnp.float32) for _ in range(NB)]
        + [pltpu.SemaphoreType.DMA for _ in range(2 * NB)]
    )
    k = functools.partial(
        pl.kernel,
        out_type=jax.ShapeDtypeStruct((n, 2 * EMBED), jnp.float32),
        mesh=mesh,
        scratch_types=scratch,
        compiler_params=pltpu.CompilerParams(use_tc_tiling_on_sc=False),
    )(_body)
    out = k(xf, table)
    return out[:, :EMBED].reshape(b, h, EMBED)
